# baseline (speedup 1.0000x reference)
import functools

import jax
import jax.numpy as jnp
from jax import lax
from jax.experimental import pallas as pl
from jax.experimental.pallas import tpu as pltpu
from jax.experimental.pallas import tpu_sc as plsc

EMB = 64
NC = 2
NS = 16
NW = NC * NS
CH = 128
SUP = 256
K = SUP // CH
NBUF = 2
LANES = 16


@functools.lru_cache(maxsize=None)
def _build(bsz, seq, vocab):
    n_total = bsz * seq
    per_w = n_total // NW
    nch = per_w // CH
    nsup = per_w // SUP
    ngrp = nsup // NBUF
    sup_per_seq = seq // SUP

    mesh = plsc.VectorSubcoreMesh(core_axis_name="c", subcore_axis_name="s")

    @functools.partial(
        pl.kernel,
        mesh=mesh,
        out_type=jax.ShapeDtypeStruct((bsz, EMB, seq), jnp.float32),
        scratch_types=[
            pltpu.VMEM((nch, CH), jnp.int32),
            pltpu.VMEM((NBUF, SUP, 128), jnp.float32),
            pltpu.VMEM((NBUF, EMB, SUP), jnp.float32),
            pltpu.SemaphoreType.DMA((NBUF,)),
            pltpu.SemaphoreType.DMA((NBUF,)),
        ],
        compiler_params=pltpu.CompilerParams(
            use_tc_tiling_on_sc=True, needs_layout_passes=False
        ),
    )
    def k(idx_hbm, table_hbm, out_hbm, idx_v, rows_v, outt_v, gsem, ssem):
        wid = lax.axis_index("s") * NC + lax.axis_index("c")
        base_sup = wid * nsup

        pltpu.sync_copy(idx_hbm.at[wid], idx_v)

        def start_gathers(t, b):
            for u in range(K):
                j = t * K + u
                pltpu.async_copy(
                    table_hbm.at[idx_v.at[j]],
                    rows_v.at[b, pl.ds(u * CH, CH)],
                    gsem.at[b],
                )

        def wait_gathers(b):
            pltpu.make_async_copy(
                table_hbm.at[pl.ds(0, SUP)], rows_v.at[b], gsem.at[b]
            ).wait()

        def start_store(t, b):
            g = base_sup + t
            pltpu.async_copy(
                outt_v.at[b],
                out_hbm.at[g // sup_per_seq, slice(None),
                           pl.ds((g % sup_per_seq) * SUP, SUP)],
                ssem.at[b],
            )

        def wait_store(b):
            pltpu.make_async_copy(
                outt_v.at[b], out_hbm.at[0, slice(None), pl.ds(0, SUP)],
                ssem.at[b],
            ).wait()

        def transpose_chunk(b):
            rows = rows_v.at[b]
            outt = outt_v.at[b]
            iota = jax.lax.iota(jnp.int32, LANES)

            def load_block(tok, e0):
                vs = []
                for d in range(LANES):
                    rot = (iota + d) & (LANES - 1)
                    vs.append((plsc.load_gather(rows, [tok, rot + e0]), rot))
                return tok, e0, vs

            def store_block(pending):
                tok, e0, vs = pending
                for v, rot in vs:
                    plsc.store_scatter(outt, [rot + e0, tok], v)

            def tg_body(tg, carry):
                tok = iota + tg * LANES
                pending = load_block(tok, 0)
                for e0 in range(LANES, EMB, LANES):
                    nxt = load_block(tok, e0)
                    store_block(pending)
                    pending = nxt
                store_block(pending)
                return carry

            lax.fori_loop(0, SUP // LANES, tg_body, 0)

        for b in range(NBUF):
            start_gathers(b, b)

        def body(g, carry):
            for b in range(NBUF):
                t = g * NBUF + b
                wait_gathers(b)

                @pl.when(g > 0)
                def _():
                    wait_store(b)

                transpose_chunk(b)
                start_store(t, b)

                @pl.when(t + NBUF < nsup)
                def _():
                    start_gathers(t + NBUF, b)

            return carry

        lax.fori_loop(0, ngrp, body, 0)

        for b in range(NBUF):
            wait_store(b)

    return k


def kernel(indices, table):
    bsz, seq = indices.shape
    n_total = bsz * seq
    vocab, emb = table.shape
    idx = indices.reshape(NW, n_total // (NW * CH), CH).astype(jnp.int32)
    table128 = jnp.pad(table, ((0, 0), (0, 128 - emb)))
    out = _build(bsz, seq, vocab)(idx, table128)
    return out.transpose(0, 2, 1)


# speedup vs baseline: 1.1004x; 1.1004x over previous
import functools

import jax
import jax.numpy as jnp
from jax import lax
from jax.experimental import pallas as pl
from jax.experimental.pallas import tpu as pltpu
from jax.experimental.pallas import tpu_sc as plsc

EMB = 64
NC = 2
NS = 16
NW = NC * NS
CH = 128
SUP = 256
K = SUP // CH
NBUF = 2
LANES = 16


@functools.lru_cache(maxsize=None)
def _build(bsz, seq, vocab):
    n_total = bsz * seq
    per_w = n_total // NW
    nch = per_w // CH
    nsup = per_w // SUP
    ngrp = nsup // NBUF
    sup_per_seq = seq // SUP

    mesh = plsc.VectorSubcoreMesh(core_axis_name="c", subcore_axis_name="s")

    @functools.partial(
        pl.kernel,
        mesh=mesh,
        out_type=jax.ShapeDtypeStruct((bsz, EMB, seq), jnp.float32),
        scratch_types=[
            pltpu.VMEM((nch, CH), jnp.int32),
            pltpu.VMEM((NBUF, SUP, 128), jnp.float32),
            pltpu.VMEM((NBUF, EMB, SUP), jnp.float32),
            pltpu.SemaphoreType.DMA((NBUF,)),
            pltpu.SemaphoreType.DMA((NBUF,)),
        ],
        compiler_params=pltpu.CompilerParams(
            use_tc_tiling_on_sc=True, needs_layout_passes=False
        ),
    )
    def k(idx_hbm, table_hbm, out_hbm, idx_v, rows_v, outt_v, gsem, ssem):
        wid = lax.axis_index("s") * NC + lax.axis_index("c")
        base_sup = wid * nsup

        pltpu.sync_copy(idx_hbm.at[wid], idx_v)

        def start_gathers(t, b):
            for u in range(K):
                j = t * K + u
                pltpu.async_copy(
                    table_hbm.at[idx_v.at[j]],
                    rows_v.at[b, pl.ds(u * CH, CH)],
                    gsem.at[b],
                )

        def wait_gathers(b):
            pltpu.make_async_copy(
                table_hbm.at[pl.ds(0, SUP)], rows_v.at[b], gsem.at[b]
            ).wait()

        def start_store(t, b):
            g = base_sup + t
            pltpu.async_copy(
                outt_v.at[b],
                out_hbm.at[g // sup_per_seq, slice(None),
                           pl.ds((g % sup_per_seq) * SUP, SUP)],
                ssem.at[b],
            )

        def wait_store(b):
            pltpu.make_async_copy(
                outt_v.at[b], out_hbm.at[0, slice(None), pl.ds(0, SUP)],
                ssem.at[b],
            ).wait()

        def transpose_chunk(b):
            rows = rows_v.at[b]
            outt = outt_v.at[b]
            iota = jax.lax.iota(jnp.int32, LANES)

            def tg_body(tg2, carry):
                for j in range(2):
                    tok = iota + (tg2 * 2 + j) * LANES
                    for e0 in range(0, EMB, LANES):
                        vs = []
                        for d in range(LANES):
                            rot = (iota + d) & (LANES - 1)
                            vs.append(
                                (plsc.load_gather(rows, [tok, rot + e0]), rot)
                            )
                        for v, rot in vs:
                            plsc.store_scatter(outt, [rot + e0, tok], v)
                return carry

            lax.fori_loop(0, SUP // (2 * LANES), tg_body, 0)

        for b in range(NBUF):
            start_gathers(b, b)

        def body(g, carry):
            for b in range(NBUF):
                t = g * NBUF + b
                wait_gathers(b)

                @pl.when(g > 0)
                def _():
                    wait_store(b)

                transpose_chunk(b)
                start_store(t, b)

                @pl.when(t + NBUF < nsup)
                def _():
                    start_gathers(t + NBUF, b)

            return carry

        lax.fori_loop(0, ngrp, body, 0)

        for b in range(NBUF):
            wait_store(b)

    return k


def kernel(indices, table):
    bsz, seq = indices.shape
    n_total = bsz * seq
    vocab, emb = table.shape
    idx = indices.reshape(NW, n_total // (NW * CH), CH).astype(jnp.int32)
    table128 = jnp.pad(table, ((0, 0), (0, 128 - emb)))
    out = _build(bsz, seq, vocab)(idx, table128)
    return out.transpose(0, 2, 1)


# speedup vs baseline: 1.1834x; 1.0755x over previous
import functools

import jax
import jax.numpy as jnp
from jax import lax
from jax.experimental import pallas as pl
from jax.experimental.pallas import tpu as pltpu
from jax.experimental.pallas import tpu_sc as plsc

EMB = 64
NC = 2
NS = 16
NW = NC * NS
CH = 128
SUP = 256
K = SUP // CH
NBUF = 2
LANES = 16


@functools.lru_cache(maxsize=None)
def _build(bsz, seq, vocab):
    n_total = bsz * seq
    per_w = n_total // NW
    nch = per_w // CH
    nsup = per_w // SUP
    ngrp = nsup // NBUF
    sup_per_seq = seq // SUP

    mesh = plsc.VectorSubcoreMesh(core_axis_name="c", subcore_axis_name="s")

    @functools.partial(
        pl.kernel,
        mesh=mesh,
        out_type=jax.ShapeDtypeStruct((bsz, EMB, seq), jnp.float32),
        scratch_types=[
            pltpu.VMEM((nch, CH), jnp.int32),
            pltpu.VMEM((NBUF, SUP, 128), jnp.float32),
            pltpu.VMEM((NBUF, EMB, SUP), jnp.float32),
            pltpu.SemaphoreType.DMA((NBUF,)),
            pltpu.SemaphoreType.DMA((NBUF,)),
        ],
        compiler_params=pltpu.CompilerParams(
            use_tc_tiling_on_sc=True, needs_layout_passes=False
        ),
    )
    def k(idx_hbm, table_hbm, out_hbm, idx_v, rows_v, outt_v, gsem, ssem):
        wid = lax.axis_index("s") * NC + lax.axis_index("c")
        base_sup = wid * nsup

        pltpu.sync_copy(idx_hbm.at[wid], idx_v)

        def start_gathers(t, b):
            for u in range(K):
                j = t * K + u
                pltpu.async_copy(
                    table_hbm.at[idx_v.at[j]],
                    rows_v.at[b, pl.ds(u * CH, CH)],
                    gsem.at[b],
                )

        def wait_gathers(b):
            pltpu.make_async_copy(
                table_hbm.at[pl.ds(0, SUP)], rows_v.at[b], gsem.at[b]
            ).wait()

        def start_store(t, b):
            g = base_sup + t
            pltpu.async_copy(
                outt_v.at[b],
                out_hbm.at[g // sup_per_seq, slice(None),
                           pl.ds((g % sup_per_seq) * SUP, SUP)],
                ssem.at[b],
            )

        def wait_store(b):
            pltpu.make_async_copy(
                outt_v.at[b], out_hbm.at[0, slice(None), pl.ds(0, SUP)],
                ssem.at[b],
            ).wait()

        def transpose_chunk(b):
            rows = rows_v.at[b]
            outt = outt_v.at[b]
            iota = jax.lax.iota(jnp.int32, LANES)

            def tg_body(tg, carry):
                tok = iota + tg * LANES
                for e0 in range(0, EMB, LANES):
                    vs = []
                    for d in range(LANES):
                        rot = (iota + d) & (LANES - 1)
                        vs.append(
                            (plsc.load_gather(rows, [tok, rot + e0]), rot)
                        )
                    for v, rot in vs:
                        plsc.store_scatter(outt, [rot + e0, tok], v)
                return carry

            lax.fori_loop(0, SUP // LANES, tg_body, 0)

        for b in range(NBUF):
            start_gathers(b, b)

        def body(g, carry):
            for b in range(NBUF):
                t = g * NBUF + b
                wait_gathers(b)

                @pl.when(g > 0)
                def _():
                    wait_store(b)

                transpose_chunk(b)
                start_store(t, b)

                @pl.when(t + NBUF < nsup)
                def _():
                    start_gathers(t + NBUF, b)

            return carry

        lax.fori_loop(0, ngrp, body, 0)

        for b in range(NBUF):
            wait_store(b)

    return k


def kernel(indices, table):
    bsz, seq = indices.shape
    n_total = bsz * seq
    vocab, emb = table.shape
    idx = indices.reshape(NW, n_total // (NW * CH), CH).astype(jnp.int32)
    table128 = jnp.pad(table, ((0, 0), (0, 128 - emb)))
    out = _build(bsz, seq, vocab)(idx, table128)
    return out.transpose(0, 2, 1)


# speedup vs baseline: 1.2218x; 1.0324x over previous
import functools

import jax
import jax.numpy as jnp
from jax import lax
from jax.experimental import pallas as pl
from jax.experimental.pallas import tpu as pltpu
from jax.experimental.pallas import tpu_sc as plsc

EMB = 64
NC = 2
NS = 16
NW = NC * NS
CH = 128
SUP = 256
K = SUP // CH
NBUF = 2
LANES = 16


@functools.lru_cache(maxsize=None)
def _build(bsz, seq, vocab):
    n_total = bsz * seq
    per_w = n_total // NW
    nch = per_w // CH
    nsup = per_w // SUP
    ngrp = nsup // NBUF
    sup_per_seq = seq // SUP

    mesh = plsc.VectorSubcoreMesh(core_axis_name="c", subcore_axis_name="s")

    @functools.partial(
        pl.kernel,
        mesh=mesh,
        out_type=jax.ShapeDtypeStruct((bsz, EMB, seq), jnp.float32),
        scratch_types=[
            pltpu.VMEM((bsz // NW, seq), jnp.int32),
            pltpu.VMEM((NBUF, SUP, 128), jnp.float32),
            pltpu.VMEM((NBUF, EMB, SUP), jnp.float32),
            pltpu.SemaphoreType.DMA((NBUF,)),
            pltpu.SemaphoreType.DMA((NBUF,)),
        ],
        compiler_params=pltpu.CompilerParams(
            use_tc_tiling_on_sc=True, needs_layout_passes=False
        ),
    )
    def k(idx_hbm, table_hbm, out_hbm, idx_v, rows_v, outt_v, gsem, ssem):
        wid = lax.axis_index("s") * NC + lax.axis_index("c")
        base_sup = wid * nsup

        rows_per_w = bsz // NW
        pltpu.sync_copy(idx_hbm.at[pl.ds(wid * rows_per_w, rows_per_w)], idx_v)

        def start_gathers(t, b):
            r = t // sup_per_seq
            l0 = (t % sup_per_seq) * SUP
            for u in range(K):
                pltpu.async_copy(
                    table_hbm.at[idx_v.at[r, pl.ds(l0 + u * CH, CH)]],
                    rows_v.at[b, pl.ds(u * CH, CH)],
                    gsem.at[b],
                )

        def wait_gathers(b):
            pltpu.make_async_copy(
                table_hbm.at[pl.ds(0, SUP)], rows_v.at[b], gsem.at[b]
            ).wait()

        def start_store(t, b):
            g = base_sup + t
            pltpu.async_copy(
                outt_v.at[b],
                out_hbm.at[g // sup_per_seq, slice(None),
                           pl.ds((g % sup_per_seq) * SUP, SUP)],
                ssem.at[b],
            )

        def wait_store(b):
            pltpu.make_async_copy(
                outt_v.at[b], out_hbm.at[0, slice(None), pl.ds(0, SUP)],
                ssem.at[b],
            ).wait()

        def transpose_chunk(b):
            rows = rows_v.at[b]
            outt = outt_v.at[b]
            iota = jax.lax.iota(jnp.int32, LANES)

            def tg_body(tg, carry):
                tok = iota + tg * LANES
                for e0 in range(0, EMB, LANES):
                    vs = []
                    for d in range(LANES):
                        rot = (iota + d) & (LANES - 1)
                        vs.append(
                            (plsc.load_gather(rows, [tok, rot + e0]), rot)
                        )
                    for v, rot in vs:
                        plsc.store_scatter(outt, [rot + e0, tok], v)
                return carry

            lax.fori_loop(0, SUP // LANES, tg_body, 0)

        for b in range(NBUF):
            start_gathers(b, b)

        def body(g, carry):
            for b in range(NBUF):
                t = g * NBUF + b
                wait_gathers(b)

                @pl.when(g > 0)
                def _():
                    wait_store(b)

                transpose_chunk(b)
                start_store(t, b)

                @pl.when(t + NBUF < nsup)
                def _():
                    start_gathers(t + NBUF, b)

            return carry

        lax.fori_loop(0, ngrp, body, 0)

        for b in range(NBUF):
            wait_store(b)

    return k


def kernel(indices, table):
    bsz, seq = indices.shape
    n_total = bsz * seq
    vocab, emb = table.shape
    table128 = jnp.pad(table, ((0, 0), (0, 128 - emb)))
    out = _build(bsz, seq, vocab)(indices.astype(jnp.int32), table128)
    return out.transpose(0, 2, 1)


# speedup vs baseline: 1.2243x; 1.0021x over previous
import functools

import jax
import jax.numpy as jnp
from jax import lax
from jax.experimental import pallas as pl
from jax.experimental.pallas import tpu as pltpu
from jax.experimental.pallas import tpu_sc as plsc

EMB = 64
NC = 2
NS = 16
NW = NC * NS
CH = 128
SUP = 256
K = SUP // CH
NBUF = 2
LANES = 16


@functools.lru_cache(maxsize=None)
def _build(bsz, seq, vocab):
    n_total = bsz * seq
    per_w = n_total // NW
    nsup = per_w // SUP
    ngrp = nsup // NBUF
    sup_per_seq = seq // SUP

    mesh = plsc.VectorSubcoreMesh(core_axis_name="c", subcore_axis_name="s")

    @functools.partial(
        pl.kernel,
        mesh=mesh,
        out_type=jax.ShapeDtypeStruct((bsz, EMB, seq), jnp.float32),
        scratch_types=[
            pltpu.VMEM((bsz // NW, seq), jnp.int32),
            pltpu.VMEM((NBUF, SUP, 128), jnp.float32),
            pltpu.VMEM((NBUF, EMB, SUP), jnp.float32),
            pltpu.SemaphoreType.DMA((NBUF,)),
            pltpu.SemaphoreType.DMA((NBUF,)),
        ],
        compiler_params=pltpu.CompilerParams(
            use_tc_tiling_on_sc=True, needs_layout_passes=False
        ),
    )
    def k(idx_hbm, table_hbm, out_hbm, idx_v, rows_v, outt_v, gsem, ssem):
        wid = lax.axis_index("s") * NC + lax.axis_index("c")
        base_sup = wid * nsup

        rows_per_w = bsz // NW
        pltpu.sync_copy(idx_hbm.at[pl.ds(wid * rows_per_w, rows_per_w)], idx_v)

        def start_gathers(t, b):
            r = t // sup_per_seq
            l0 = (t % sup_per_seq) * SUP
            for u in range(K):
                pltpu.async_copy(
                    table_hbm.at[idx_v.at[r, pl.ds(l0 + u * CH, CH)]],
                    rows_v.at[b, pl.ds(u * CH, CH)],
                    gsem.at[b],
                )

        def wait_gathers(b):
            pltpu.make_async_copy(
                table_hbm.at[pl.ds(0, SUP)], rows_v.at[b], gsem.at[b]
            ).wait()

        def start_store(t, b):
            g = base_sup + t
            pltpu.async_copy(
                outt_v.at[b],
                out_hbm.at[g // sup_per_seq, slice(None),
                           pl.ds((g % sup_per_seq) * SUP, SUP)],
                ssem.at[b],
            )

        def wait_store(b):
            pltpu.make_async_copy(
                outt_v.at[b], out_hbm.at[0, slice(None), pl.ds(0, SUP)],
                ssem.at[b],
            ).wait()

        def transpose_chunk(b):
            rows = rows_v.at[b]
            outt = outt_v.at[b]
            iota = jax.lax.iota(jnp.int32, LANES)

            def tg_body(tg, carry):
                tok = iota + tg * LANES
                for e0 in range(0, EMB, LANES):
                    vs = []
                    for d in range(LANES):
                        rot = (iota + d) & (LANES - 1)
                        vs.append(
                            (plsc.load_gather(rows, [tok, rot + e0]), rot)
                        )
                    for v, rot in vs:
                        plsc.store_scatter(outt, [rot + e0, tok], v)
                return carry

            lax.fori_loop(0, SUP // LANES, tg_body, 0)

        for b in range(NBUF):
            start_gathers(b, b)

        def body(g, carry):
            for b in range(NBUF):
                t = g * NBUF + b
                wait_gathers(b)

                @pl.when(g > 0)
                def _():
                    wait_store(b)

                transpose_chunk(b)
                start_store(t, b)

                @pl.when(t + NBUF < nsup)
                def _():
                    start_gathers(t + NBUF, b)

            return carry

        lax.fori_loop(0, ngrp, body, 0)

        for b in range(NBUF):
            wait_store(b)

    return k


def kernel(indices, table):
    bsz, seq = indices.shape
    n_total = bsz * seq
    vocab, emb = table.shape
    table128 = jnp.pad(table, ((0, 0), (0, 128 - emb)))
    out = _build(bsz, seq, vocab)(indices.astype(jnp.int32), table128)
    return out.transpose(0, 2, 1)
